# baseline (speedup 1.0000x reference)
import jax
import jax.numpy as jnp
from jax import lax
from jax.experimental import pallas as pl
from jax.experimental.pallas import tpu as pltpu

_MAX_ATOMS = 13
_T = 512


def _grouped_matmul_body(lohi_ref, ids_ref, rn_ref, w_ref, b_ref, out_ref):
    out_ref[...] = jnp.zeros_like(out_ref)
    lo = lohi_ref[0, 0, 0]
    hi = lohi_ref[0, 0, 1]

    def body(p, carry):
        m = (ids_ref[0] == p).astype(jnp.float32)
        contrib = jnp.dot(rn_ref[...] * m, w_ref[p],
                          preferred_element_type=jnp.float32)
        out_ref[...] += contrib + m * b_ref[p]
        return carry

    lax.fori_loop(lo, hi + 1, body, 0)
    out_ref[...] = jnp.maximum(out_ref[...], 0.0)


def kernel(x, nodes, fact, fact_dim, params, bias):
    F = fact.shape[0]
    N, L = nodes.shape
    P, _, R = params.shape
    fact = fact.astype(jnp.int32)

    ids = (x[fact[:, 0], 1].astype(jnp.int32) * _MAX_ATOMS
           + x[fact[:, 0], 2].astype(jnp.int32))
    order = jnp.argsort(ids)
    order2 = jnp.argsort(ids + 1)
    ids_s = ids[order2]

    rows = 2 * F
    rtot = pl.cdiv(rows, _T) * _T
    pad = rtot - rows
    tiles = rtot // _T

    idx_rows = fact[order].ravel()
    rn = jnp.pad(nodes[idx_rows], ((0, pad), (0, 0)))

    ids_rep = jnp.repeat(ids_s, 2)
    ids_p = jnp.pad(ids_rep, (0, pad), constant_values=P - 1)
    lohi = jnp.stack([ids_p[::_T], ids_p[_T - 1::_T]],
                     axis=1).astype(jnp.int32).reshape(tiles, 1, 2)
    ids_b = ids_p.reshape(tiles, _T, 1)

    out_sorted = pl.pallas_call(
        _grouped_matmul_body,
        grid=(tiles,),
        in_specs=[
            pl.BlockSpec((1, 1, 2), lambda i: (i, 0, 0), memory_space=pltpu.SMEM),
            pl.BlockSpec((1, _T, 1), lambda i: (i, 0, 0)),
            pl.BlockSpec((_T, L), lambda i: (i, 0)),
            pl.BlockSpec((P, L, R), lambda i: (0, 0, 0)),
            pl.BlockSpec((P, 1, R), lambda i: (0, 0, 0)),
        ],
        out_specs=pl.BlockSpec((_T, R), lambda i: (i, 0)),
        out_shape=jax.ShapeDtypeStruct((rtot, R), jnp.float32),
    )(lohi, ids_b, rn, params, bias)

    inv = jnp.zeros((F,), jnp.int32).at[order].set(jnp.arange(F, dtype=jnp.int32))
    src_rows = (2 * inv[None, :] + jnp.arange(2, dtype=jnp.int32)[:, None]).ravel()
    return out_sorted[src_rows].reshape(2, F, R)


# speedup vs baseline: 1.2779x; 1.2779x over previous
import jax
import jax.numpy as jnp
from jax import lax
from jax.experimental import pallas as pl
from jax.experimental.pallas import tpu as pltpu
from jax.experimental.pallas import tpu_sc as plsc

_MAX_ATOMS = 13
_T = 512
_NC = 2
_NS = 16
_NW = _NC * _NS
_IW = 80


def _grouped_matmul_body(lohi_ref, ids_ref, rn_ref, w_ref, out_ref):
    out_ref[...] = jnp.zeros_like(out_ref)
    lo = lohi_ref[0, 0, 0]
    hi = lohi_ref[0, 0, 1]

    def body(p, carry):
        m = (ids_ref[0] == p).astype(jnp.float32)
        out_ref[...] += jnp.dot(rn_ref[...] * m, w_ref[p],
                                preferred_element_type=jnp.float32)
        return carry

    lax.fori_loop(lo, hi + 1, body, 0)
    out_ref[...] = jnp.maximum(out_ref[...], 0.0)


def kernel(x, nodes, fact, fact_dim, params, bias):
    F = fact.shape[0]
    N, L = nodes.shape
    P, _, R = params.shape
    fact = fact.astype(jnp.int32)

    ids = (x[fact[:, 0], 1].astype(jnp.int32) * _MAX_ATOMS
           + x[fact[:, 0], 2].astype(jnp.int32))
    order = jnp.argsort(ids)

    cbj = pl.cdiv(pl.cdiv(F, _NW), _IW) * _IW
    rtot = _NW * 2 * cbj
    tiles = rtot // _T
    nck = cbj // _IW
    assert rtot % _T == 0

    order_pad = jnp.pad(order, (0, _NW * cbj - F), mode="edge")
    fact0 = fact[:, 0]
    fact1 = fact[:, 1]
    nodes_aug = jnp.concatenate(
        [nodes, jnp.ones((N, 1), nodes.dtype),
         jnp.zeros((N, R - L - 1), nodes.dtype)], axis=1)
    w_aug = jnp.concatenate(
        [params, bias, jnp.zeros((P, R - L - 1, R), params.dtype)], axis=1)
    mesh = plsc.VectorSubcoreMesh(core_axis_name="c", subcore_axis_name="s")

    def sc_gather(order_hbm, fact0_hbm, fact1_hbm, ids_hbm, nodes_hbm,
                  rn_hbm, idsrep_hbm,
                  o_v, f0_v, f1_v, idv_v, rows0_v, rows1_v, s0, s1, s2):
        wid = lax.axis_index("s") * _NC + lax.axis_index("c")
        jbase = wid * cbj
        pltpu.sync_copy(order_hbm.at[pl.ds(jbase, cbj)], o_v)
        c0 = pltpu.async_copy(fact0_hbm.at[o_v], f0_v, s0)
        c1 = pltpu.async_copy(fact1_hbm.at[o_v], f1_v, s1)
        c2 = pltpu.async_copy(ids_hbm.at[o_v], idv_v, s2)
        c0.wait()
        c1.wait()
        c2.wait()
        d0 = pltpu.async_copy(nodes_hbm.at[f0_v], rows0_v, s0)
        d1 = pltpu.async_copy(nodes_hbm.at[f1_v], rows1_v, s1)
        d0.wait()
        d1.wait()
        base = 2 * jbase
        pltpu.sync_copy(rows0_v, rn_hbm.at[pl.ds(base, cbj)])
        pltpu.sync_copy(rows1_v, rn_hbm.at[pl.ds(base + cbj, cbj)])
        pltpu.sync_copy(idv_v, idsrep_hbm.at[pl.ds(base, cbj)])
        pltpu.sync_copy(idv_v, idsrep_hbm.at[pl.ds(base + cbj, cbj)])

    rn, ids_rep = pl.kernel(
        sc_gather,
        out_type=(jax.ShapeDtypeStruct((rtot, R), jnp.float32),
                  jax.ShapeDtypeStruct((rtot,), jnp.int32)),
        mesh=mesh,
        scratch_types=[
            pltpu.VMEM((cbj,), jnp.int32),
            pltpu.VMEM((cbj,), jnp.int32),
            pltpu.VMEM((cbj,), jnp.int32),
            pltpu.VMEM((cbj,), jnp.int32),
            pltpu.VMEM((cbj, R), jnp.float32),
            pltpu.VMEM((cbj, R), jnp.float32),
            pltpu.SemaphoreType.DMA,
            pltpu.SemaphoreType.DMA,
            pltpu.SemaphoreType.DMA,
        ],
    )(order_pad, fact0, fact1, ids, nodes_aug)

    idt = ids_rep.reshape(tiles, _T)
    lohi = jnp.stack([idt.min(axis=1), idt.max(axis=1)],
                     axis=1).reshape(tiles, 1, 2)
    ids_b = ids_rep.reshape(tiles, _T, 1)

    out_sorted = pl.pallas_call(
        _grouped_matmul_body,
        grid=(tiles,),
        in_specs=[
            pl.BlockSpec((1, 1, 2), lambda i: (i, 0, 0), memory_space=pltpu.SMEM),
            pl.BlockSpec((1, _T, 1), lambda i: (i, 0, 0)),
            pl.BlockSpec((_T, R), lambda i: (i, 0)),
            pl.BlockSpec((P, R, R), lambda i: (0, 0, 0)),
        ],
        out_specs=pl.BlockSpec((_T, R), lambda i: (i, 0)),
        out_shape=jax.ShapeDtypeStruct((rtot, R), jnp.float32),
    )(lohi, ids_b, rn, w_aug)

    dst0 = order_pad.reshape(_NW * nck, _IW)
    dst1 = (order_pad + F).reshape(_NW * nck, _IW)

    def sc_unsort(outs_hbm, dst0_hbm, dst1_hbm, final_hbm,
                  d0_v, d1_v, rows_v, sem):
        wid = lax.axis_index("s") * _NC + lax.axis_index("c")
        jbase = wid * cbj
        pltpu.sync_copy(dst0_hbm.at[pl.ds(wid * nck, nck)], d0_v)
        pltpu.sync_copy(dst1_hbm.at[pl.ds(wid * nck, nck)], d1_v)
        pltpu.sync_copy(outs_hbm.at[pl.ds(2 * jbase, 2 * cbj)], rows_v)
        for c in range(nck):
            pltpu.async_copy(rows_v.at[pl.ds(c * _IW, _IW)],
                             final_hbm.at[d0_v.at[c]], sem)
            pltpu.async_copy(rows_v.at[pl.ds(cbj + c * _IW, _IW)],
                             final_hbm.at[d1_v.at[c]], sem)
        for c in range(nck):
            pltpu.make_async_copy(rows_v.at[pl.ds(c * _IW, _IW)],
                                  final_hbm.at[d0_v.at[c]], sem).wait()
            pltpu.make_async_copy(rows_v.at[pl.ds(cbj + c * _IW, _IW)],
                                  final_hbm.at[d1_v.at[c]], sem).wait()

    final = pl.kernel(
        sc_unsort,
        out_type=jax.ShapeDtypeStruct((2 * F, R), jnp.float32),
        mesh=mesh,
        scratch_types=[
            pltpu.VMEM((nck, _IW), jnp.int32),
            pltpu.VMEM((nck, _IW), jnp.int32),
            pltpu.VMEM((2 * cbj, R), jnp.float32),
            pltpu.SemaphoreType.DMA,
        ],
    )(out_sorted, dst0, dst1)

    return final.reshape(2, F, R)
